# Initial kernel scaffold; baseline (speedup 1.0000x reference)
import jax
import jax.numpy as jnp
from jax.experimental import pallas as pl


def kernel(x, table, pos_emb):
    raise NotImplementedError("write your pallas kernel here")


# speedup vs baseline: 1.6074x; 1.6074x over previous
import functools
import math

import jax
import jax.numpy as jnp
from jax import lax
from jax.experimental import pallas as pl
from jax.experimental.pallas import tpu as pltpu
from jax.experimental.pallas import tpu_sc as plsc

_NC = 2
_NS = 16
_NW = _NC * _NS
_LANES = 16
_CHUNK = 128


def _build_sc_lookup(seq, batch, d):
    rows = seq * batch
    rpw = rows // _NW
    ppw = seq // _NW
    n_chunks = rpw // _CHUNK
    scale = float(math.sqrt(d))
    mesh = plsc.VectorSubcoreMesh(core_axis_name="c", subcore_axis_name="s")

    @functools.partial(
        pl.kernel,
        mesh=mesh,
        out_type=jax.ShapeDtypeStruct((rows, d), jnp.float32),
        scratch_types=[
            pltpu.VMEM((n_chunks, _CHUNK), jnp.int32),
            pltpu.VMEM((rpw, d), jnp.float32),
            pltpu.VMEM((ppw, d), jnp.float32),
            pltpu.SemaphoreType.DMA,
        ],
    )
    def sc_lookup(table_hbm, idx_hbm, pos_hbm, out_hbm, idx_v, rows_v, pos_v, sem):
        wid = lax.axis_index("s") * _NC + lax.axis_index("c")
        pltpu.sync_copy(idx_hbm.at[wid], idx_v)
        copies = [
            pltpu.async_copy(
                table_hbm.at[idx_v.at[j]],
                rows_v.at[pl.ds(j * _CHUNK, _CHUNK)],
                sem,
            )
            for j in range(n_chunks)
        ]
        pltpu.sync_copy(pos_hbm.at[pl.ds(wid * ppw, ppw)], pos_v)
        for cp in copies:
            cp.wait()

        def step(p, carry):
            pos_regs = [pos_v[p, pl.ds(k * _LANES, _LANES)] for k in range(d // _LANES)]
            for b in range(batch):
                r = p * batch + b
                for k in range(d // _LANES):
                    sl = pl.ds(k * _LANES, _LANES)
                    rows_v[r, sl] = rows_v[r, sl] * scale + pos_regs[k]
            return carry

        lax.fori_loop(0, ppw, step, 0)
        pltpu.sync_copy(rows_v, out_hbm.at[pl.ds(wid * rpw, rpw)])

    return sc_lookup


def kernel(x, table, pos_emb):
    seq, batch = x.shape
    d = table.shape[1]
    rows = seq * batch
    rpw = rows // _NW
    idx3 = x.reshape(_NW, rpw // _CHUNK, _CHUNK)
    pos2 = pos_emb[:seq].reshape(seq, d)
    out = _build_sc_lookup(seq, batch, d)(table, idx3, pos2)
    return out.reshape(seq, batch, d)
